# baseline (speedup 1.0000x reference)
import functools

import jax
import jax.numpy as jnp
from jax import lax
from jax.experimental import pallas as pl
from jax.experimental.pallas import tpu as pltpu
from jax.experimental.pallas import tpu_sc as plsc

B, N, E = 4, 10000, 160000
D_IN, D1, D2, D3, D_OUT = 128, 64, 128, 256, 512

NP = 10240
TRASH = N
EP = 163840
K = 128
RT = NP // 16
EPT = EP // 16
EPT2 = EP // 32
RB = 1024
NB = NP // RB

_MESH = dict(core_axis_name="c", subcore_axis_name="s")
_SC_PARAMS = pltpu.CompilerParams(use_tc_tiling_on_sc=False)


GK2 = 512
NG2 = EPT2 // GK2


def _make_deg():
    mesh = plsc.VectorSubcoreMesh(**_MESH)

    @functools.partial(
        pl.kernel,
        out_type=jax.ShapeDtypeStruct((2, B, NP), jnp.float32),
        mesh=mesh,
        compiler_params=_SC_PARAMS,
        scratch_types=[
            pltpu.VMEM_SHARED((NP,), jnp.float32),
            pltpu.VMEM((RT,), jnp.float32),
            pltpu.VMEM((GK2,), jnp.float32),
            pltpu.VMEM((GK2,), jnp.int32),
            pltpu.VMEM((GK2,), jnp.int32),
            pltpu.SemaphoreType.DMA, pltpu.SemaphoreType.DMA,
            pltpu.SemaphoreType.DMA, pltpu.SemaphoreType.DMA,
            pltpu.SemaphoreType.DMA,
        ],
    )
    def k(dst_hbm, cnt_hbm, deg_sh, zbuf, ones_v, didx0, didx1,
          isem0, isem1, ssem0, ssem1, wsem):
        c = lax.axis_index("c")
        s = lax.axis_index("s")
        didx = [didx0, didx1]
        isem = [isem0, isem1]
        ssem = [ssem0, ssem1]

        @pl.loop(0, RT, step=16)
        def _(i):
            zbuf[pl.ds(i, 16)] = jnp.zeros((16,), jnp.float32)

        @pl.loop(0, GK2, step=16)
        def _(i):
            ones_v[pl.ds(i, 16)] = jnp.ones((16,), jnp.float32)

        @pl.loop(0, B)
        def _(b):
            def load_idx(g, p):
                pltpu.async_copy(dst_hbm.at[b, c, s, g], didx[p], isem[p])

            def wait_idx(p):
                pltpu.make_async_copy(dst_hbm.at[b, c, s, 0], didx[p],
                                      isem[p]).wait()

            def start_scatters(p):
                pltpu.async_copy(ones_v, deg_sh.at[didx[p]], ssem[p],
                                 add=True)

            def wait_scatters(p):
                pltpu.make_async_copy(ones_v, deg_sh.at[didx[p]],
                                      ssem[p]).wait()

            init = pltpu.async_copy(zbuf, deg_sh.at[pl.ds(s * RT, RT)], wsem)
            load_idx(0, 0)
            load_idx(1, 1)
            init.wait()
            plsc.subcore_barrier()

            @pl.loop(0, (NG2 - 2) // 2)
            def _(t):
                g = 2 * t
                wait_idx(0)
                start_scatters(0)
                wait_idx(1)
                start_scatters(1)
                wait_scatters(0)
                load_idx(g + 2, 0)
                wait_scatters(1)
                load_idx(g + 3, 1)

            wait_idx(0)
            start_scatters(0)
            wait_idx(1)
            start_scatters(1)
            wait_scatters(0)
            wait_scatters(1)
            plsc.subcore_barrier()
            pltpu.sync_copy(deg_sh.at[pl.ds(s * RT, RT)],
                            cnt_hbm.at[c, b, pl.ds(s * RT, RT)])

    return k


_deg_kernel = _make_deg()


def _make_scatter(W, bv, hsb):
    GK = 512
    NG = EPT2 // GK
    mesh = plsc.VectorSubcoreMesh(**_MESH)

    @functools.partial(
        pl.kernel,
        out_type=jax.ShapeDtypeStruct((2, NP, W), jnp.bfloat16),
        mesh=mesh,
        compiler_params=_SC_PARAMS,
        scratch_types=[
            pltpu.VMEM_SHARED((NP, W), jnp.bfloat16),
            pltpu.VMEM((GK,), jnp.int32), pltpu.VMEM((GK,), jnp.int32),
            pltpu.VMEM((GK,), jnp.int32), pltpu.VMEM((GK,), jnp.int32),
            pltpu.VMEM((GK, W), jnp.bfloat16),
            pltpu.VMEM((GK, W), jnp.bfloat16),
            pltpu.SemaphoreType.DMA, pltpu.SemaphoreType.DMA,
            pltpu.SemaphoreType.DMA, pltpu.SemaphoreType.DMA,
            pltpu.SemaphoreType.DMA, pltpu.SemaphoreType.DMA,
            pltpu.SemaphoreType.DMA,
        ],
    )
    def k(hs_hbm, zeros_hbm, srcv, dstv, out_hbm, acc_sh,
          sidx0, sidx1, didx0, didx1, rows0, rows1,
          isem0, isem1, gsem0, gsem1, ssem0, ssem1, wsem):
        c = lax.axis_index("c")
        s = lax.axis_index("s")
        sidx = [sidx0, sidx1]
        didx = [didx0, didx1]
        rows = [rows0, rows1]
        isem = [isem0, isem1]
        gsem = [gsem0, gsem1]
        ssem = [ssem0, ssem1]

        def load_idx(g, p):
            pltpu.async_copy(srcv.at[bv, c, s, g], sidx[p], isem[p])
            pltpu.async_copy(dstv.at[bv, c, s, g], didx[p], isem[p])

        def start_gathers(p):
            pltpu.make_async_copy(srcv.at[bv, c, s, 0], sidx[p],
                                  isem[p]).wait()
            pltpu.make_async_copy(dstv.at[bv, c, s, 0], didx[p],
                                  isem[p]).wait()
            pltpu.async_copy(hs_hbm.at[hsb].at[sidx[p]], rows[p], gsem[p])

        def wait_gathers(p):
            pltpu.make_async_copy(hs_hbm.at[hsb].at[sidx[p]], rows[p],
                                  gsem[p]).wait()

        def start_scatters(p):
            pltpu.async_copy(rows[p], acc_sh.at[didx[p]], ssem[p],
                             add=True)

        def wait_scatters(p):
            pltpu.make_async_copy(rows[p], acc_sh.at[didx[p]],
                                  ssem[p]).wait()

        @pl.when(c == 0)
        def _():
            pltpu.async_copy(hs_hbm.at[hsb, pl.ds(s * RT, RT)],
                             acc_sh.at[pl.ds(s * RT, RT)], wsem)

        @pl.when(c == 1)
        def _():
            pltpu.async_copy(zeros_hbm, acc_sh.at[pl.ds(s * RT, RT)], wsem)

        load_idx(0, 0)
        load_idx(1, 1)
        start_gathers(0)
        pltpu.make_async_copy(zeros_hbm, acc_sh.at[pl.ds(s * RT, RT)],
                              wsem).wait()
        plsc.subcore_barrier()

        @pl.loop(0, (NG - 2) // 2)
        def _(t):
            g = 2 * t
            wait_gathers(0)
            start_scatters(0)
            start_gathers(1)
            wait_scatters(0)
            load_idx(g + 2, 0)
            wait_gathers(1)
            start_scatters(1)
            start_gathers(0)
            wait_scatters(1)
            load_idx(g + 3, 1)

        wait_gathers(0)
        start_scatters(0)
        start_gathers(1)
        wait_scatters(0)
        wait_gathers(1)
        start_scatters(1)
        wait_scatters(1)
        plsc.subcore_barrier()
        pltpu.sync_copy(acc_sh.at[pl.ds(s * RT, RT)],
                        out_hbm.at[c, pl.ds(s * RT, RT)])

    def call(hs_flat, srcv, dstv):
        zeros = jnp.zeros((RT, W), jnp.bfloat16)
        return k(hs_flat.reshape(-1, NP, W), zeros, srcv, dstv)

    return call


_scatter_l1 = [_make_scatter(64, b, b) for b in range(B)]
_scatter_l2 = [_make_scatter(64, b, 0) for b in range(B)]
_scatter_l3 = [_make_scatter(128, b, 0) for b in range(B)]


def _tc1_body(x_ref, w_ref, cnt_ref, hs_ref, dis_ref):
    deg = cnt_ref[0, 0] + cnt_ref[1, 0] + 1.0
    dis = lax.rsqrt(deg)
    h = jnp.dot(x_ref[0], w_ref[...], preferred_element_type=jnp.float32)
    hs_ref[0] = (h * dis).astype(jnp.bfloat16)
    dis_ref[0] = dis


def _tc1_call(x_p, W1, cnt2):
    return pl.pallas_call(
        _tc1_body,
        grid=(B, NB),
        in_specs=[
            pl.BlockSpec((1, RB, D_IN), lambda b, n: (b, n, 0)),
            pl.BlockSpec((D_IN, D1), lambda b, n: (0, 0)),
            pl.BlockSpec((2, 1, RB, 1), lambda b, n: (0, b, n, 0)),
        ],
        out_specs=[
            pl.BlockSpec((1, RB, D1), lambda b, n: (b, n, 0)),
            pl.BlockSpec((1, RB, 1), lambda b, n: (b, n, 0)),
        ],
        out_shape=[
            jax.ShapeDtypeStruct((B, NP, D1), jnp.bfloat16),
            jax.ShapeDtypeStruct((B, NP, 1), jnp.float32),
        ],
    )(x_p, W1, cnt2)


def _tc2_body(acc_ref, dis_ref, b_ref, out_ref):
    dis = dis_ref[0]
    bias = b_ref[...]
    agg = (acc_ref[0].astype(jnp.float32) + acc_ref[1].astype(jnp.float32))
    g = jnp.maximum(agg * dis + bias[None, :], 0.0)
    out_ref[...] = (g * dis).astype(jnp.bfloat16)


def _tc2_call(bv, acc, dis, b1):
    return pl.pallas_call(
        _tc2_body,
        grid=(NB,),
        in_specs=[
            pl.BlockSpec((2, RB, D1), lambda n: (0, n, 0)),
            pl.BlockSpec((1, RB, 1), lambda n: (bv, n, 0)),
            pl.BlockSpec((D1,), lambda n: (0,)),
        ],
        out_specs=pl.BlockSpec((RB, D1), lambda n: (n, 0)),
        out_shape=jax.ShapeDtypeStruct((NP, D1), jnp.bfloat16),
    )(acc, dis, b1)


def _tc3_body(acc_ref, dis_ref, w_ref, b_ref, out_ref):
    dis = dis_ref[0]
    bias = b_ref[...]
    agg = (acc_ref[0].astype(jnp.float32) + acc_ref[1].astype(jnp.float32))
    h = jnp.dot(agg, w_ref[...], preferred_element_type=jnp.float32)
    g = jnp.maximum(h * dis + bias[None, :], 0.0)
    out_ref[...] = (g * dis).astype(jnp.bfloat16)


def _tc3_call(bv, acc, dis, W2, b2):
    return pl.pallas_call(
        _tc3_body,
        grid=(NB,),
        in_specs=[
            pl.BlockSpec((2, RB, D1), lambda n: (0, n, 0)),
            pl.BlockSpec((1, RB, 1), lambda n: (bv, n, 0)),
            pl.BlockSpec((D1, D2), lambda n: (0, 0)),
            pl.BlockSpec((D2,), lambda n: (0,)),
        ],
        out_specs=pl.BlockSpec((RB, D2), lambda n: (n, 0)),
        out_shape=jax.ShapeDtypeStruct((NP, D2), jnp.bfloat16),
    )(acc, dis, W2, b2)


def _tc4_body(acc_ref, dis_ref, w3_ref, b_ref, wfc_ref, bfc_ref, out_ref,
              pool_scr):
    n = pl.program_id(0)
    dis = dis_ref[0]
    bias = b_ref[...]
    agg = (acc_ref[0].astype(jnp.float32) + acc_ref[1].astype(jnp.float32))
    h = jnp.dot(agg, w3_ref[...], preferred_element_type=jnp.float32)
    g = jnp.maximum(h * dis + bias[None, :], 0.0)
    rowid = lax.broadcasted_iota(jnp.int32, (RB, 1), 0) + n * RB
    m = (rowid < N).astype(jnp.float32)
    p = jnp.sum(g * m, axis=0)[None, :]

    @pl.when(n == 0)
    def _():
        pool_scr[...] = p

    @pl.when(n > 0)
    def _():
        pool_scr[...] = pool_scr[...] + p

    @pl.when(n == NB - 1)
    def _():
        pooled = pool_scr[...] * (1.0 / N)
        res = (jnp.dot(pooled, wfc_ref[...],
                       preferred_element_type=jnp.float32)
               + bfc_ref[...][None, :])
        out_ref[...] = jnp.broadcast_to(res, (8, D_OUT))


def _tc4_call(bv, acc, dis, W3, b3, Wfc, bfc):
    return pl.pallas_call(
        _tc4_body,
        grid=(NB,),
        in_specs=[
            pl.BlockSpec((2, RB, D2), lambda n: (0, n, 0)),
            pl.BlockSpec((1, RB, 1), lambda n: (bv, n, 0)),
            pl.BlockSpec((D2, D3), lambda n: (0, 0)),
            pl.BlockSpec((D3,), lambda n: (0,)),
            pl.BlockSpec((D3, D_OUT), lambda n: (0, 0)),
            pl.BlockSpec((D_OUT,), lambda n: (0,)),
        ],
        out_specs=pl.BlockSpec((8, D_OUT), lambda n: (0, 0)),
        out_shape=jax.ShapeDtypeStruct((8, D_OUT), jnp.float32),
        scratch_shapes=[pltpu.VMEM((1, D3), jnp.float32)],
    )(acc, dis, W3, b3, Wfc, bfc)


def kernel(x, edge_index, W1, b1, W2, b2, W3, b3, Wfc, bfc):
    src = edge_index[:, 0, :].astype(jnp.int32)
    dst = edge_index[:, 1, :].astype(jnp.int32)
    src_p = jnp.pad(src, ((0, 0), (0, EP - E)))
    dst_p = jnp.pad(dst, ((0, 0), (0, EP - E)), constant_values=TRASH)
    x_p = jnp.pad(x, ((0, 0), (0, NP - N), (0, 0)))
    srcv = src_p.reshape(B, 2, 16, EPT2 // 512, 512)
    dstv = dst_p.reshape(B, 2, 16, EPT2 // 512, 512)

    cnt2 = _deg_kernel(dstv).reshape(2, B, NP, 1)
    hs1, dis = _tc1_call(x_p, W1, cnt2)
    hs1f = hs1.reshape(B * NP, D1)
    outs = []
    for b in range(B):
        agg1 = _scatter_l1[b](hs1f, srcv, dstv)
        z2 = _tc2_call(b, agg1, dis, b1)
        agg2 = _scatter_l2[b](z2.reshape(NP, D1), srcv, dstv)
        z3 = _tc3_call(b, agg2, dis, W2, b2)
        agg3 = _scatter_l3[b](z3.reshape(NP, D2), srcv, dstv)
        outs.append(_tc4_call(b, agg3, dis, W3, b3, Wfc, bfc)[0:1])
    return jnp.concatenate(outs, axis=0)


# speedup vs baseline: 1.3647x; 1.3647x over previous
import functools

import jax
import jax.numpy as jnp
from jax import lax
from jax.experimental import pallas as pl
from jax.experimental.pallas import tpu as pltpu
from jax.experimental.pallas import tpu_sc as plsc

B, N, E = 4, 10000, 160000
D_IN, D1, D2, D3, D_OUT = 128, 64, 128, 256, 512

NP = 10240
TRASH = N
EP = 163840
K = 128
RT = NP // 16
EPT = EP // 16
EPT2 = EP // 32
RB = 1024
NB = NP // RB

_MESH = dict(core_axis_name="c", subcore_axis_name="s")
_SC_PARAMS = pltpu.CompilerParams(use_tc_tiling_on_sc=False)


G2 = 4
GK2 = G2 * K
NG2 = EPT2 // GK2


def _deg_call(dst_p):
    mesh = plsc.VectorSubcoreMesh(**_MESH)
    dstv = dst_p.reshape(B, 2, 16, NG2, G2, K)

    @functools.partial(
        pl.kernel,
        out_type=jax.ShapeDtypeStruct((2, B, NP), jnp.float32),
        mesh=mesh,
        compiler_params=_SC_PARAMS,
        scratch_types=[
            pltpu.VMEM_SHARED((NP,), jnp.float32),
            pltpu.VMEM((RT,), jnp.float32),
            pltpu.VMEM((K,), jnp.float32),
            pltpu.VMEM((G2, K), jnp.int32),
            pltpu.VMEM((G2, K), jnp.int32),
            pltpu.SemaphoreType.DMA, pltpu.SemaphoreType.DMA,
            pltpu.SemaphoreType.DMA, pltpu.SemaphoreType.DMA,
            pltpu.SemaphoreType.DMA,
        ],
    )
    def k(dst_hbm, cnt_hbm, deg_sh, zbuf, ones_v, didx0, didx1,
          isem0, isem1, ssem0, ssem1, wsem):
        c = lax.axis_index("c")
        s = lax.axis_index("s")
        didx = [didx0, didx1]
        isem = [isem0, isem1]
        ssem = [ssem0, ssem1]

        @pl.loop(0, RT, step=16)
        def _(i):
            zbuf[pl.ds(i, 16)] = jnp.zeros((16,), jnp.float32)

        @pl.loop(0, K, step=16)
        def _(i):
            ones_v[pl.ds(i, 16)] = jnp.ones((16,), jnp.float32)

        @pl.loop(0, B)
        def _(b):
            def load_idx(g, p):
                pltpu.async_copy(dst_hbm.at[b, c, s, g], didx[p], isem[p])

            def wait_idx(p):
                pltpu.make_async_copy(dst_hbm.at[b, c, s, 0], didx[p],
                                      isem[p]).wait()

            def start_scatters(p):
                for j in range(G2):
                    pltpu.async_copy(ones_v, deg_sh.at[didx[p].at[j]],
                                     ssem[p], add=True)

            def wait_scatters(p):
                for j in range(G2):
                    pltpu.make_async_copy(ones_v, deg_sh.at[didx[p].at[j]],
                                          ssem[p]).wait()

            init = pltpu.async_copy(zbuf, deg_sh.at[pl.ds(s * RT, RT)], wsem)
            load_idx(0, 0)
            load_idx(1, 1)
            init.wait()
            plsc.subcore_barrier()

            @pl.loop(0, (NG2 - 2) // 2)
            def _(t):
                g = 2 * t
                wait_idx(0)
                start_scatters(0)
                wait_idx(1)
                start_scatters(1)
                wait_scatters(0)
                load_idx(g + 2, 0)
                wait_scatters(1)
                load_idx(g + 3, 1)

            wait_idx(0)
            start_scatters(0)
            wait_idx(1)
            start_scatters(1)
            wait_scatters(0)
            wait_scatters(1)
            plsc.subcore_barrier()
            pltpu.sync_copy(deg_sh.at[pl.ds(s * RT, RT)],
                            cnt_hbm.at[c, b, pl.ds(s * RT, RT)])

    return k(dstv)


def _make_scatter(W, bv, hsb):
    GK = 1024
    NG = EPT // GK
    mesh = plsc.VectorSubcoreMesh(**_MESH)

    @functools.partial(
        pl.kernel,
        out_type=jax.ShapeDtypeStruct((2, NP, W), jnp.bfloat16),
        mesh=mesh,
        compiler_params=_SC_PARAMS,
        scratch_types=[
            pltpu.VMEM_SHARED((NP, W), jnp.bfloat16),
            pltpu.VMEM((GK,), jnp.int32), pltpu.VMEM((GK,), jnp.int32),
            pltpu.VMEM((GK,), jnp.int32), pltpu.VMEM((GK,), jnp.int32),
            pltpu.VMEM((GK, W), jnp.bfloat16),
            pltpu.VMEM((GK, W), jnp.bfloat16),
            pltpu.SemaphoreType.DMA, pltpu.SemaphoreType.DMA,
            pltpu.SemaphoreType.DMA, pltpu.SemaphoreType.DMA,
            pltpu.SemaphoreType.DMA, pltpu.SemaphoreType.DMA,
            pltpu.SemaphoreType.DMA,
        ],
    )
    def k(hs_hbm, srcv, dstv, out_hbm, acc_sh,
          sidx0, sidx1, didx0, didx1, rows0, rows1,
          isem0, isem1, gsem0, gsem1, ssem0, ssem1, wsem):
        c = lax.axis_index("c")
        s = lax.axis_index("s")
        sidx = [sidx0, sidx1]
        didx = [didx0, didx1]
        rows = [rows0, rows1]
        isem = [isem0, isem1]
        gsem = [gsem0, gsem1]
        ssem = [ssem0, ssem1]
        bc = hsb + c

        def load_idx(g, p):
            pltpu.async_copy(srcv.at[bv, s, g], sidx[p], isem[p])
            pltpu.async_copy(dstv.at[bv, s, g], didx[p], isem[p])

        def start_gathers(p):
            pltpu.make_async_copy(srcv.at[bv, s, 0], sidx[p],
                                  isem[p]).wait()
            pltpu.make_async_copy(dstv.at[bv, s, 0], didx[p],
                                  isem[p]).wait()
            pltpu.async_copy(hs_hbm.at[bc].at[sidx[p]], rows[p], gsem[p])

        def wait_gathers(p):
            pltpu.make_async_copy(hs_hbm.at[bc].at[sidx[p]], rows[p],
                                  gsem[p]).wait()

        def start_scatters(p):
            pltpu.async_copy(rows[p], acc_sh.at[didx[p]], ssem[p],
                             add=True)

        def wait_scatters(p):
            pltpu.make_async_copy(rows[p], acc_sh.at[didx[p]],
                                  ssem[p]).wait()

        init = pltpu.async_copy(hs_hbm.at[bc, pl.ds(s * RT, RT)],
                                acc_sh.at[pl.ds(s * RT, RT)], wsem)
        load_idx(0, 0)
        load_idx(1, 1)
        start_gathers(0)
        init.wait()
        plsc.subcore_barrier()

        @pl.loop(0, (NG - 2) // 2)
        def _(t):
            g = 2 * t
            wait_gathers(0)
            start_scatters(0)
            start_gathers(1)
            wait_scatters(0)
            load_idx(g + 2, 0)
            wait_gathers(1)
            start_scatters(1)
            start_gathers(0)
            wait_scatters(1)
            load_idx(g + 3, 1)

        wait_gathers(0)
        start_scatters(0)
        start_gathers(1)
        wait_scatters(0)
        wait_gathers(1)
        start_scatters(1)
        wait_scatters(1)
        plsc.subcore_barrier()
        pltpu.sync_copy(acc_sh.at[pl.ds(s * RT, RT)],
                        out_hbm.at[c, pl.ds(s * RT, RT)])

    def call(hs_flat, srcv, dstv):
        return k(hs_flat.reshape(-1, NP, W), srcv, dstv)

    return call


_scatter_l1 = [_make_scatter(32, b, 2 * b) for b in range(B)]
_scatter_l2 = [_make_scatter(32, b, 0) for b in range(B)]
_scatter_l3 = [_make_scatter(64, b, 0) for b in range(B)]


def _tc1_body(x_ref, w_ref, cnt_ref, hs_ref, dis_ref):
    deg = cnt_ref[0, 0] + cnt_ref[1, 0] + 1.0
    dis = lax.rsqrt(deg)
    h = jnp.dot(x_ref[0], w_ref[...], preferred_element_type=jnp.float32)
    hs = (h * dis).astype(jnp.bfloat16)
    w_out = D1 // 2
    for i in range(2):
        hs_ref[0, i] = hs[:, i * w_out:(i + 1) * w_out]
    dis_ref[0] = dis


def _tc1_call(x_p, W1, cnt2):
    return pl.pallas_call(
        _tc1_body,
        grid=(B, NB),
        in_specs=[
            pl.BlockSpec((1, RB, D_IN), lambda b, n: (b, n, 0)),
            pl.BlockSpec((D_IN, D1), lambda b, n: (0, 0)),
            pl.BlockSpec((2, 1, RB, 1), lambda b, n: (0, b, n, 0)),
        ],
        out_specs=[
            pl.BlockSpec((1, 2, RB, D1 // 2), lambda b, n: (b, 0, n, 0)),
            pl.BlockSpec((1, RB, 1), lambda b, n: (b, n, 0)),
        ],
        out_shape=[
            jax.ShapeDtypeStruct((B, 2, NP, D1 // 2), jnp.bfloat16),
            jax.ShapeDtypeStruct((B, NP, 1), jnp.float32),
        ],
    )(x_p, W1, cnt2)


def _tc2_body(acc_ref, dis_ref, b_ref, out_ref):
    dis = dis_ref[0]
    bias = b_ref[...]
    for i in range(2):
        g = jnp.maximum(acc_ref[i].astype(jnp.float32) * dis
                        + bias[i * 32:(i + 1) * 32][None, :], 0.0)
        out_ref[i] = (g * dis).astype(jnp.bfloat16)


def _tc2_call(bv, acc, dis, b1):
    return pl.pallas_call(
        _tc2_body,
        grid=(NB,),
        in_specs=[
            pl.BlockSpec((2, RB, 32), lambda n: (0, n, 0)),
            pl.BlockSpec((1, RB, 1), lambda n: (bv, n, 0)),
            pl.BlockSpec((D1,), lambda n: (0,)),
        ],
        out_specs=pl.BlockSpec((2, RB, 32), lambda n: (0, n, 0)),
        out_shape=jax.ShapeDtypeStruct((2, NP, 32), jnp.bfloat16),
    )(acc, dis, b1)


def _tc3_body(acc_ref, dis_ref, w_ref, b_ref, out_ref):
    dis = dis_ref[0]
    bias = b_ref[...]
    w = w_ref[...]
    h = None
    for i in range(2):
        hi = jnp.dot(acc_ref[i].astype(jnp.float32),
                     w[i * 32:(i + 1) * 32],
                     preferred_element_type=jnp.float32)
        h = hi if h is None else h + hi
    g = jnp.maximum(h * dis + bias[None, :], 0.0)
    z = (g * dis).astype(jnp.bfloat16)
    for i in range(2):
        out_ref[i] = z[:, i * 64:(i + 1) * 64]


def _tc3_call(bv, acc, dis, W2, b2):
    return pl.pallas_call(
        _tc3_body,
        grid=(NB,),
        in_specs=[
            pl.BlockSpec((2, RB, 32), lambda n: (0, n, 0)),
            pl.BlockSpec((1, RB, 1), lambda n: (bv, n, 0)),
            pl.BlockSpec((D1, D2), lambda n: (0, 0)),
            pl.BlockSpec((D2,), lambda n: (0,)),
        ],
        out_specs=pl.BlockSpec((2, RB, 64), lambda n: (0, n, 0)),
        out_shape=jax.ShapeDtypeStruct((2, NP, 64), jnp.bfloat16),
    )(acc, dis, W2, b2)


def _tc4_body(acc_ref, dis_ref, w3_ref, b_ref, wfc_ref, bfc_ref, out_ref,
              pool_scr):
    n = pl.program_id(0)
    dis = dis_ref[0]
    bias = b_ref[...]
    w3 = w3_ref[...]
    h = None
    for i in range(2):
        hi = jnp.dot(acc_ref[i].astype(jnp.float32),
                     w3[i * 64:(i + 1) * 64],
                     preferred_element_type=jnp.float32)
        h = hi if h is None else h + hi
    g = jnp.maximum(h * dis + bias[None, :], 0.0)
    rowid = lax.broadcasted_iota(jnp.int32, (RB, 1), 0) + n * RB
    m = (rowid < N).astype(jnp.float32)
    p = jnp.sum(g * m, axis=0)[None, :]

    @pl.when(n == 0)
    def _():
        pool_scr[...] = p

    @pl.when(n > 0)
    def _():
        pool_scr[...] = pool_scr[...] + p

    @pl.when(n == NB - 1)
    def _():
        pooled = pool_scr[...] * (1.0 / N)
        res = (jnp.dot(pooled, wfc_ref[...],
                       preferred_element_type=jnp.float32)
               + bfc_ref[...][None, :])
        out_ref[...] = jnp.broadcast_to(res, (8, D_OUT))


def _tc4_call(bv, acc, dis, W3, b3, Wfc, bfc):
    return pl.pallas_call(
        _tc4_body,
        grid=(NB,),
        in_specs=[
            pl.BlockSpec((2, RB, 64), lambda n: (0, n, 0)),
            pl.BlockSpec((1, RB, 1), lambda n: (bv, n, 0)),
            pl.BlockSpec((D2, D3), lambda n: (0, 0)),
            pl.BlockSpec((D3,), lambda n: (0,)),
            pl.BlockSpec((D3, D_OUT), lambda n: (0, 0)),
            pl.BlockSpec((D_OUT,), lambda n: (0,)),
        ],
        out_specs=pl.BlockSpec((8, D_OUT), lambda n: (0, 0)),
        out_shape=jax.ShapeDtypeStruct((8, D_OUT), jnp.float32),
        scratch_shapes=[pltpu.VMEM((1, D3), jnp.float32)],
    )(acc, dis, W3, b3, Wfc, bfc)


def kernel(x, edge_index, W1, b1, W2, b2, W3, b3, Wfc, bfc):
    src = edge_index[:, 0, :].astype(jnp.int32)
    dst = edge_index[:, 1, :].astype(jnp.int32)
    src_p = jnp.pad(src, ((0, 0), (0, EP - E)))
    dst_p = jnp.pad(dst, ((0, 0), (0, EP - E)), constant_values=TRASH)
    x_p = jnp.pad(x, ((0, 0), (0, NP - N), (0, 0)))
    srcv = src_p.reshape(B, 16, EPT // 1024, 1024)
    dstv = dst_p.reshape(B, 16, EPT // 1024, 1024)

    cnt2 = _deg_call(dst_p).reshape(2, B, NP, 1)
    hs1, dis = _tc1_call(x_p, W1, cnt2)
    hs1f = hs1.reshape(B * 2 * NP, 32)
    outs = []
    for b in range(B):
        agg1 = _scatter_l1[b](hs1f, srcv, dstv)
        z2 = _tc2_call(b, agg1, dis, b1)
        agg2 = _scatter_l2[b](z2.reshape(2 * NP, 32), srcv, dstv)
        z3 = _tc3_call(b, agg2, dis, W2, b2)
        agg3 = _scatter_l3[b](z3.reshape(2 * NP, 64), srcv, dstv)
        outs.append(_tc4_call(b, agg3, dis, W3, b3, Wfc, bfc)[0:1])
    return jnp.concatenate(outs, axis=0)
